# baseline (speedup 1.0000x reference)
import jax
import jax.numpy as jnp
from jax.experimental import pallas as pl
from jax.experimental.pallas import tpu as pltpu

B, N = 2, 128
EPOS = 1024
A0, U0, F0 = 128, 128, 32
M0 = 32
C = A0 + U0 + F0
KB = 8
MB = M0 // KB


def _erf(x):
    a1, a2, a3, a4, a5 = (0.254829592, -0.284496736, 1.421413741,
                          -1.453152027, 1.061405429)
    p = 0.3275911
    sgn = jnp.sign(x)
    ax = jnp.abs(x)
    t = 1.0 / (1.0 + p * ax)
    poly = ((((a5 * t + a4) * t + a3) * t + a2) * t + a1) * t
    return sgn * (1.0 - poly * jnp.exp(-ax * ax))


def _gelu(x):
    return 0.5 * x * (1.0 + _erf(x * 0.7071067811865476))


def _fused_kernel(a_ref, basesT_ref, wbases_ref, ei_ref,
                  fa_W_ref, fa_b_ref, ff_W_ref, ff_b_ref, fu_W_ref, fu_b_ref,
                  k1W_ref, k1b_ref, k2W_ref, k2b_ref, root_ref,
                  s0_wt_ref, s0_wW_ref, s0_wb_ref, s0_fcW_ref, s0_fcb_ref,
                  fc1_W1_ref, fc1_b1_ref, fc1_W2_ref, fc1_b2_ref,
                  out_ref,
                  xhat_s, xN_s, u_s, x1acc_s):
    f32 = jnp.float32
    step = pl.program_id(0)

    @pl.when(step == 0)
    def _front():
        a = a_ref[...]
        grid2 = a[:, :, 1:3]

        fin = jnp.concatenate([jnp.ones((B, N, 1), f32), grid2], axis=-1)
        f = (fin.reshape(B * N, 3) @ ff_W_ref[...].T
             + ff_b_ref[...]).reshape(B, N, F0)
        av = (a.reshape(B * N, 3) @ fa_W_ref[...].T
              + fa_b_ref[...]).reshape(B, N, A0)
        u = (jnp.concatenate([av, f], axis=-1).reshape(B * N, A0 + F0)
             @ fu_W_ref[...].T + fu_b_ref[...]).reshape(B, N, U0)

        avT = jnp.transpose(av, (0, 2, 1))
        x_all = jnp.concatenate([avT, u], axis=-1).reshape(B * N, A0 + U0)
        x0 = x_all[:N]
        pw0 = jnp.concatenate([avT[0], u[0], grid2[0]], axis=-1)
        k1W = k1W_ref[...]
        ga = pw0 @ k1W[:, : A0 + U0 + 2].T
        gb = pw0 @ k1W[:, A0 + U0 + 2:].T
        z = x0 @ k2W_ref[...]
        xb = x0 @ k2b_ref[...]
        table = jnp.concatenate([z, xb, ga], axis=-1)

        iota_n = jax.lax.broadcasted_iota(jnp.int32, (EPOS, N), 1)
        ei = ei_ref[...]
        oh_src = (ei[0] == iota_n).astype(f32)
        oh_dst = (ei[1] == iota_n).astype(f32)

        gath = oh_src @ table
        zg = gath[:, : 8 * F0]
        xbg = gath[:, 8 * F0: 8 * F0 + F0]
        gag = gath[:, 8 * F0 + F0:]
        gbg = oh_dst @ gb
        h = _gelu(gag + gbg + k1b_ref[...])

        msg = xbg
        for r in range(8):
            msg = msg + h[:, r:r + 1] * zg[:, r * F0:(r + 1) * F0]

        s = jax.lax.dot_general(oh_dst, msg, (((0,), (0,)), ((), ())))
        cnt = jnp.sum(oh_dst, axis=0)
        mean = s / jnp.maximum(cnt, 1.0)[:, None]
        rootc = x_all @ root_ref[...]
        mean_full = jnp.concatenate([mean, jnp.zeros((N, F0), f32)], axis=0)
        gp = (mean_full + rootc).reshape(B, N, F0)
        df = f - gp

        xN = jnp.concatenate([avT, u, df], axis=-1)
        x_hat = jax.lax.dot_general(xN, wbases_ref[...],
                                    (((1,), (0,)), ((), ())))
        xhat_s[...] = jnp.transpose(x_hat, (2, 0, 1))
        xN_s[...] = xN
        u_s[...] = u
        x1acc_s[...] = jnp.zeros((B, C, N), f32)

    xh_blk = xhat_s[pl.ds(step * MB, MB)].astype(jnp.bfloat16)
    xh2_blk = jax.lax.dot_general(xh_blk, s0_wt_ref[...],
                                  (((2,), (1,)), ((0,), (0,))),
                                  preferred_element_type=f32)
    bT_blk = basesT_ref[pl.ds(step * MB, MB), :]
    x1acc_s[...] += jax.lax.dot_general(xh2_blk, bT_blk,
                                        (((0,), (0,)), ((), ())))

    @pl.when(step == KB - 1)
    def _tail():
        xN = xN_s[...]
        u = u_s[...]
        x1N = jnp.transpose(x1acc_s[...], (0, 2, 1))
        x2N = (xN.reshape(B * N, C) @ s0_wW_ref[...].T
               + s0_wb_ref[...]).reshape(B, N, C)
        xnew = xN + _gelu(x1N + x2N)
        un = u + (xnew.reshape(B * N, C) @ s0_fcW_ref[...].T
                  + s0_fcb_ref[...]).reshape(B, N, U0)

        hd = _gelu(un.reshape(B * N, U0) @ fc1_W1_ref[...].T + fc1_b1_ref[...])
        W2p = jnp.concatenate([fc1_W2_ref[...],
                               jnp.zeros((127, 2 * U0), f32)], axis=0)
        b2p = jnp.concatenate([fc1_b2_ref[...], jnp.zeros((127,), f32)])
        out = hd @ W2p.T + b2p
        out_ref[...] = out.reshape(B, N, 128)


def kernel(a, bases, wbases, ei_pos, ei_pro, fc0_a_W, fc0_a_b, fc0_f_W,
           fc0_f_b, fc0_u_W, fc0_u_b, pos_k1W, pos_k1b, pos_k2W, pos_k2b,
           pos_root, s0_w, s0_wW, s0_wb, s0_fcW, s0_fcb, s1_w, s1_wW,
           s1_wb, s1_fcW, s1_fcb, pro_k1W, pro_k1b, pro_k2W, pro_k2b,
           pro_root, fc1_W1, fc1_b1, fc1_W2, fc1_b2, *, interpret=False):
    del ei_pro, s1_w, s1_wW, s1_wb, s1_fcW, s1_fcb
    del pro_k1W, pro_k1b, pro_k2W, pro_k2b, pro_root

    ei32 = ei_pos.astype(jnp.int32)
    K2 = pos_k2W.reshape(A0 + U0, F0, 8).transpose(0, 2, 1).reshape(A0 + U0, 8 * F0)
    s0_wt = s0_w.transpose(2, 0, 1).astype(jnp.bfloat16)
    basesT = bases.T

    f32 = jnp.float32
    full = lambda shp: pl.BlockSpec(shp, lambda i, _n=None: (0,) * len(shp))
    in_specs = [
        full((B, N, 3)),
        full((M0, N)),
        full((N, M0)),
        full((2, EPOS, 1)),
        full((A0, 3)), full((A0,)),
        full((F0, 3)), full((F0,)),
        full((U0, A0 + F0)), full((U0,)),
        full((8, 2 * (A0 + U0 + 2))), full((8,)),
        full((A0 + U0, 8 * F0)), full((A0 + U0, F0)),
        full((A0 + U0, F0)),
        pl.BlockSpec((MB, C, C), lambda i: (i, 0, 0)),
        full((C, C)), full((C,)),
        full((U0, C)), full((U0,)),
        full((2 * U0, U0)), full((2 * U0,)),
        full((1, 2 * U0)), full((1,)),
    ]

    out = pl.pallas_call(
        _fused_kernel,
        grid=(KB,),
        in_specs=in_specs,
        out_specs=pl.BlockSpec((B, N, 128), lambda i: (0, 0, 0)),
        out_shape=jax.ShapeDtypeStruct((B, N, 128), f32),
        scratch_shapes=[
            pltpu.VMEM((M0, B, C), f32),
            pltpu.VMEM((B, N, C), f32),
            pltpu.VMEM((B, N, U0), f32),
            pltpu.VMEM((B, C, N), f32),
        ],
        interpret=interpret,
    )(a, basesT, wbases, ei32,
      fc0_a_W, fc0_a_b, fc0_f_W, fc0_f_b, fc0_u_W, fc0_u_b,
      pos_k1W, pos_k1b, K2, pos_k2b.reshape(A0 + U0, F0), pos_root,
      s0_wt, s0_wW, s0_wb, s0_fcW, s0_fcb,
      fc1_W1, fc1_b1, fc1_W2, fc1_b2)
    return out[:, :, :1]


# speedup vs baseline: 1.0624x; 1.0624x over previous
import jax
import jax.numpy as jnp
from jax.experimental import pallas as pl
from jax.experimental.pallas import tpu as pltpu

B, N = 2, 128
EPOS = 1024
A0, U0, F0 = 128, 128, 32
M0 = 32
C = A0 + U0 + F0
KB = 4
MB = M0 // KB


def _erf(x):
    a1, a2, a3, a4, a5 = (0.254829592, -0.284496736, 1.421413741,
                          -1.453152027, 1.061405429)
    p = 0.3275911
    sgn = jnp.sign(x)
    ax = jnp.abs(x)
    t = 1.0 / (1.0 + p * ax)
    poly = ((((a5 * t + a4) * t + a3) * t + a2) * t + a1) * t
    return sgn * (1.0 - poly * jnp.exp(-ax * ax))


def _gelu(x):
    return 0.5 * x * (1.0 + _erf(x * 0.7071067811865476))


def _fused_kernel(a_ref, basesT_ref, wbases_ref, ei_ref,
                  fa_W_ref, fa_b_ref, ff_W_ref, ff_b_ref, fu_W_ref, fu_b_ref,
                  k1W_ref, k1b_ref, k2W_ref, k2b_ref, root_ref,
                  s0_wt_ref, s0_wW_ref, s0_wb_ref, s0_fcW_ref, s0_fcb_ref,
                  fc1_W1_ref, fc1_b1_ref, fc1_W2_ref, fc1_b2_ref,
                  out_ref,
                  xhat_s, xN_s, u_s, x1acc_s):
    f32 = jnp.float32
    step = pl.program_id(0)

    @pl.when(step == 0)
    def _front():
        a = a_ref[...]
        grid2 = a[:, :, 1:3]

        fin = jnp.concatenate([jnp.ones((B, N, 1), f32), grid2], axis=-1)
        f = (fin.reshape(B * N, 3) @ ff_W_ref[...].T
             + ff_b_ref[...]).reshape(B, N, F0)
        av = (a.reshape(B * N, 3) @ fa_W_ref[...].T
              + fa_b_ref[...]).reshape(B, N, A0)
        u = (jnp.concatenate([av, f], axis=-1).reshape(B * N, A0 + F0)
             @ fu_W_ref[...].T + fu_b_ref[...]).reshape(B, N, U0)

        avT = jnp.transpose(av, (0, 2, 1))
        x_all = jnp.concatenate([avT, u], axis=-1).reshape(B * N, A0 + U0)
        x0 = x_all[:N]
        pw0 = jnp.concatenate([avT[0], u[0], grid2[0]], axis=-1)
        k1W = k1W_ref[...]
        ga = pw0 @ k1W[:, : A0 + U0 + 2].T
        gb = pw0 @ k1W[:, A0 + U0 + 2:].T
        z = x0 @ k2W_ref[...]
        xb = x0 @ k2b_ref[...]
        table = jnp.concatenate([z, xb, ga], axis=-1)

        iota_n = jax.lax.broadcasted_iota(jnp.int32, (EPOS, N), 1)
        ei = ei_ref[...]
        oh_src = (ei[0] == iota_n).astype(f32)
        oh_dst = (ei[1] == iota_n).astype(f32)

        gath = oh_src @ table
        zg = gath[:, : 8 * F0]
        xbg = gath[:, 8 * F0: 8 * F0 + F0]
        gag = gath[:, 8 * F0 + F0:]
        gbg = oh_dst @ gb
        h = _gelu(gag + gbg + k1b_ref[...])

        msg = xbg
        for r in range(8):
            msg = msg + h[:, r:r + 1] * zg[:, r * F0:(r + 1) * F0]

        s = jax.lax.dot_general(oh_dst, msg, (((0,), (0,)), ((), ())))
        cnt = jnp.sum(oh_dst, axis=0)
        mean = s / jnp.maximum(cnt, 1.0)[:, None]
        rootc = x_all @ root_ref[...]
        mean_full = jnp.concatenate([mean, jnp.zeros((N, F0), f32)], axis=0)
        gp = (mean_full + rootc).reshape(B, N, F0)
        df = f - gp

        xN = jnp.concatenate([avT, u, df], axis=-1)
        x_hat = jax.lax.dot_general(xN, wbases_ref[...],
                                    (((1,), (0,)), ((), ())))
        xhat_s[...] = jnp.transpose(x_hat, (2, 0, 1))
        xN_s[...] = xN
        u_s[...] = u
        x1acc_s[...] = jnp.zeros((B, C, N), f32)

    xh_blk = xhat_s[pl.ds(step * MB, MB)].astype(jnp.bfloat16)
    xh2_blk = jax.lax.dot_general(xh_blk, s0_wt_ref[...],
                                  (((2,), (1,)), ((0,), (0,))),
                                  preferred_element_type=f32)
    bT_blk = basesT_ref[pl.ds(step * MB, MB), :]
    x1acc_s[...] += jax.lax.dot_general(xh2_blk, bT_blk,
                                        (((0,), (0,)), ((), ())))

    @pl.when(step == KB - 1)
    def _tail():
        xN = xN_s[...]
        u = u_s[...]
        x1N = jnp.transpose(x1acc_s[...], (0, 2, 1))
        x2N = (xN.reshape(B * N, C) @ s0_wW_ref[...].T
               + s0_wb_ref[...]).reshape(B, N, C)
        xnew = xN + _gelu(x1N + x2N)
        un = u + (xnew.reshape(B * N, C) @ s0_fcW_ref[...].T
                  + s0_fcb_ref[...]).reshape(B, N, U0)

        hd = _gelu(un.reshape(B * N, U0) @ fc1_W1_ref[...].T + fc1_b1_ref[...])
        W2p = jnp.concatenate([fc1_W2_ref[...],
                               jnp.zeros((127, 2 * U0), f32)], axis=0)
        b2p = jnp.concatenate([fc1_b2_ref[...], jnp.zeros((127,), f32)])
        out = hd @ W2p.T + b2p
        out_ref[...] = out.reshape(B, N, 128)


def kernel(a, bases, wbases, ei_pos, ei_pro, fc0_a_W, fc0_a_b, fc0_f_W,
           fc0_f_b, fc0_u_W, fc0_u_b, pos_k1W, pos_k1b, pos_k2W, pos_k2b,
           pos_root, s0_w, s0_wW, s0_wb, s0_fcW, s0_fcb, s1_w, s1_wW,
           s1_wb, s1_fcW, s1_fcb, pro_k1W, pro_k1b, pro_k2W, pro_k2b,
           pro_root, fc1_W1, fc1_b1, fc1_W2, fc1_b2, *, interpret=False):
    del ei_pro, s1_w, s1_wW, s1_wb, s1_fcW, s1_fcb
    del pro_k1W, pro_k1b, pro_k2W, pro_k2b, pro_root

    ei32 = ei_pos.astype(jnp.int32)
    K2 = pos_k2W.reshape(A0 + U0, F0, 8).transpose(0, 2, 1).reshape(A0 + U0, 8 * F0)
    s0_wt = s0_w.transpose(2, 0, 1).astype(jnp.bfloat16)
    basesT = bases.T

    f32 = jnp.float32
    full = lambda shp: pl.BlockSpec(shp, lambda i, _n=None: (0,) * len(shp))
    in_specs = [
        full((B, N, 3)),
        full((M0, N)),
        full((N, M0)),
        full((2, EPOS, 1)),
        full((A0, 3)), full((A0,)),
        full((F0, 3)), full((F0,)),
        full((U0, A0 + F0)), full((U0,)),
        full((8, 2 * (A0 + U0 + 2))), full((8,)),
        full((A0 + U0, 8 * F0)), full((A0 + U0, F0)),
        full((A0 + U0, F0)),
        pl.BlockSpec((MB, C, C), lambda i: (i, 0, 0)),
        full((C, C)), full((C,)),
        full((U0, C)), full((U0,)),
        full((2 * U0, U0)), full((2 * U0,)),
        full((1, 2 * U0)), full((1,)),
    ]

    out = pl.pallas_call(
        _fused_kernel,
        grid=(KB,),
        in_specs=in_specs,
        out_specs=pl.BlockSpec((B, N, 128), lambda i: (0, 0, 0)),
        out_shape=jax.ShapeDtypeStruct((B, N, 128), f32),
        scratch_shapes=[
            pltpu.VMEM((M0, B, C), f32),
            pltpu.VMEM((B, N, C), f32),
            pltpu.VMEM((B, N, U0), f32),
            pltpu.VMEM((B, C, N), f32),
        ],
        interpret=interpret,
    )(a, basesT, wbases, ei32,
      fc0_a_W, fc0_a_b, fc0_f_W, fc0_f_b, fc0_u_W, fc0_u_b,
      pos_k1W, pos_k1b, K2, pos_k2b.reshape(A0 + U0, F0), pos_root,
      s0_wt, s0_wW, s0_wb, s0_fcW, s0_fcb,
      fc1_W1, fc1_b1, fc1_W2, fc1_b2)
    return out[:, :, :1]
